# baseline (speedup 1.0000x reference)
import functools

import jax
import jax.numpy as jnp
from jax import lax
from jax.experimental import pallas as pl
from jax.experimental.pallas import tpu as pltpu
from jax.experimental.pallas import tpu_sc as plsc

N = 10000
E = 320000
D = 128
LN_EPS = 1e-5

_BLK = 1000
_C = 40
_NW = 32
_EPT = E // _NW
_NCHUNK = _EPT // _C
_NZC = N // _C
_PLEN = 144
_AW = D // 2
_RL = 2 * _AW


def _prep_body(x_ref, w1_ref, b1_ref, g1_ref, bb1_ref, we1t_ref, we1b_ref,
               be1_ref, a_ref, b_ref, h_ref):
    h0 = jnp.dot(x_ref[...], w1_ref[...],
                 preferred_element_type=jnp.float32) + b1_ref[...]
    m = jnp.mean(h0, axis=-1, keepdims=True)
    v = jnp.mean((h0 - m) ** 2, axis=-1, keepdims=True)
    h = jnp.maximum(
        (h0 - m) / jnp.sqrt(v + LN_EPS) * g1_ref[...] + bb1_ref[...], 0.0)
    a_ref[...] = jnp.dot(h, we1t_ref[...],
                         preferred_element_type=jnp.float32)
    h_ref[...] = h
    b_ref[...] = jnp.dot(h, we1b_ref[...],
                         preferred_element_type=jnp.float32) + be1_ref[...]


def _final_body(h_ref, p_ref, wgt_ref, wgb_ref, bg_ref, gf_ref, bf_ref,
                o_ref):
    h = h_ref[...]
    aggr = p_ref[0] + p_ref[1]
    g = (jnp.dot(h, wgt_ref[...], preferred_element_type=jnp.float32)
         + jnp.dot(aggr, wgb_ref[...], preferred_element_type=jnp.float32)
         + bg_ref[...])
    gate = 1.0 / (1.0 + jnp.exp(-g))
    hn = gate * aggr + (1.0 - gate) * h
    m = jnp.mean(hn, axis=-1, keepdims=True)
    v = jnp.mean((hn - m) ** 2, axis=-1, keepdims=True)
    o_ref[...] = (hn - m) / jnp.sqrt(v + LN_EPS) * gf_ref[...] + bf_ref[...]


@functools.partial(
    pl.kernel,
    mesh=plsc.VectorSubcoreMesh(core_axis_name="c", subcore_axis_name="s"),
    out_type=jax.ShapeDtypeStruct((2, N, D), jnp.float32),
    scratch_types=[
        pltpu.VMEM((2, _C), jnp.int32),
        pltpu.VMEM((2, _C), jnp.int32),
        pltpu.VMEM((2, _C), jnp.int32),
        pltpu.VMEM((_C, _RL), jnp.float32),
        pltpu.VMEM((_C, _RL), jnp.float32),
        pltpu.VMEM((_C, D), jnp.float32),
        pltpu.VMEM((_C, D), jnp.float32),
        pltpu.VMEM((_C, D), jnp.float32),
        pltpu.VMEM((_C, D), jnp.float32),
        pltpu.VMEM((_PLEN,), jnp.float32),
        pltpu.VMEM((_C + 16,), jnp.float32),
        pltpu.VMEM((_C,), jnp.float32),
        pltpu.VMEM_SHARED((N, D), jnp.float32),
        pltpu.SemaphoreType.DMA,
        pltpu.SemaphoreType.DMA,
        pltpu.SemaphoreType.DMA,
        pltpu.SemaphoreType.DMA,
        pltpu.SemaphoreType.DMA,
        pltpu.SemaphoreType.DMA,
    ],
)
def _edge_kernel(row_hbm, col_hbm, ah_hbm, b_hbm, par_hbm, out_hbm,
                 rowv, colv, scol, ah0, ah1, bb0, bb1, sc0, sc1, par_buf,
                 zpack, wbuf, aggr, ga0, ga1, gb0, gb1, ss0, ss1):
    cid = lax.axis_index("c")
    sid = lax.axis_index("s")
    wid = cid * 16 + sid
    ah = (ah0, ah1)
    bb = (bb0, bb1)
    sc = (sc0, sc1)
    ga = (ga0, ga1)
    gb = (gb0, gb1)
    ss = (ss0, ss1)

    pltpu.sync_copy(par_hbm, par_buf)

    zv = jnp.zeros((16,), jnp.float32)

    def _zrow(r, carry):
        for j in range(D // 16):
            sc0[r, pl.ds(16 * j, 16)] = zv
        return carry

    lax.fori_loop(0, _C, _zrow, 0)
    for k in range((_NZC + 15) // 16):
        ci = sid + 16 * k

        @pl.when(ci < _NZC)
        def _():
            pltpu.sync_copy(sc0, aggr.at[pl.ds(ci * _C, _C)])

    plsc.subcore_barrier()

    e0 = wid * _EPT

    def _issue_gather(p, kk):
        base = e0 + kk * _C
        pltpu.sync_copy(row_hbm.at[pl.ds(base, _C)], rowv.at[p])
        pltpu.sync_copy(col_hbm.at[pl.ds(base, _C)], colv.at[p])
        pltpu.async_copy(ah_hbm.at[rowv.at[p]], ah[p], ga[p])
        pltpu.async_copy(b_hbm.at[colv.at[p]], bb[p], gb[p])

    def _compute(p):
        ahp, bbp, scp = ah[p], bb[p], sc[p]
        lanes = lax.iota(jnp.int32, 16)
        m0 = lanes == 0

        himask = jnp.full((16,), -65536, jnp.int32)

        @plsc.parallel_loop(0, _C, unroll=20)
        def _edge(e):
            acc = par_buf[pl.ds(D, 16)]
            for m in range(_AW // 16):
                aw = lax.bitcast_convert_type(
                    ahp[e, pl.ds(16 * m, 16)], jnp.int32)
                ae = lax.bitcast_convert_type(aw << 16, jnp.float32)
                ao = lax.bitcast_convert_type(aw & himask, jnp.float32)
                bev = bbp[e, pl.ds(16 * (2 * m), 16)]
                bod = bbp[e, pl.ds(16 * (2 * m + 1), 16)]
                t0 = jnp.maximum(ae + bev, 0.0)
                acc = acc + t0 * par_buf[pl.ds(16 * (2 * m), 16)]
                t1 = jnp.maximum(ao + bod, 0.0)
                acc = acc + t1 * par_buf[pl.ds(16 * (2 * m + 1), 16)]
            for sh in (8, 4, 2, 1):
                acc = acc + acc.at[lanes ^ sh].get(mode="promise_in_bounds")
            wv = 1.0 / (1.0 + jnp.exp(-acc))
            for m in range(_AW // 16):
                hw = lax.bitcast_convert_type(
                    ahp[e, pl.ds(_AW + 16 * m, 16)], jnp.int32)
                he = lax.bitcast_convert_type(hw << 16, jnp.float32)
                ho = lax.bitcast_convert_type(hw & himask, jnp.float32)
                scp[e, pl.ds(16 * (2 * m), 16)] = he * wv
                scp[e, pl.ds(16 * (2 * m + 1), 16)] = ho * wv

    _issue_gather(0, 0)
    _issue_gather(1, 1)

    def _outer(i, carry):
        for p in (0, 1):
            k = 2 * i + p

            @pl.when(k >= 2)
            def _():
                pltpu.make_async_copy(sc[p], aggr.at[scol.at[p]],
                                      ss[p]).wait()

            pltpu.make_async_copy(ah_hbm.at[rowv.at[p]], ah[p], ga[p]).wait()
            pltpu.make_async_copy(b_hbm.at[colv.at[p]], bb[p], gb[p]).wait()
            _compute(p)
            for off in (0, 16, 24):
                scol[p, pl.ds(off, 16)] = colv[p, pl.ds(off, 16)]
            pltpu.async_copy(sc[p], aggr.at[scol.at[p]], ss[p], add=True)

            @pl.when(k + 2 < _NCHUNK)
            def _():
                _issue_gather(p, k + 2)
        return carry

    lax.fori_loop(0, _NCHUNK // 2, _outer, 0)
    for p in (0, 1):
        pltpu.make_async_copy(sc[p], aggr.at[scol.at[p]], ss[p]).wait()

    plsc.subcore_barrier()
    for k in range((_NZC + 15) // 16):
        ci = sid + 16 * k

        @pl.when(ci < _NZC)
        def _():
            r0 = ci * _C
            pltpu.sync_copy(aggr.at[pl.ds(r0, _C)],
                            out_hbm.at[cid, pl.ds(r0, _C)])


def kernel(x, edge_index, W1, b1, g1, bb1, We1, be1, We2, be2,
           Wn1, bn1, Wn2, bn2, Wg, bg, gf, bf):
    row = edge_index[0].astype(jnp.int32)
    col = edge_index[1].astype(jnp.int32)
    r1 = lambda a: a.reshape(1, D)

    av, bv, hv = pl.pallas_call(
        _prep_body,
        grid=(N // _BLK,),
        in_specs=[
            pl.BlockSpec((_BLK, D), lambda i: (i, 0)),
            pl.BlockSpec((D, D), lambda i: (0, 0)),
            pl.BlockSpec((1, D), lambda i: (0, 0)),
            pl.BlockSpec((1, D), lambda i: (0, 0)),
            pl.BlockSpec((1, D), lambda i: (0, 0)),
            pl.BlockSpec((D, D), lambda i: (0, 0)),
            pl.BlockSpec((D, D), lambda i: (0, 0)),
            pl.BlockSpec((1, D), lambda i: (0, 0)),
        ],
        out_specs=[
            pl.BlockSpec((_BLK, D), lambda i: (i, 0)),
            pl.BlockSpec((_BLK, D), lambda i: (i, 0)),
            pl.BlockSpec((_BLK, D), lambda i: (i, 0)),
        ],
        out_shape=[
            jax.ShapeDtypeStruct((N, D), jnp.float32),
            jax.ShapeDtypeStruct((N, D), jnp.float32),
            jax.ShapeDtypeStruct((N, D), jnp.float32),
        ],
    )(x, W1, r1(b1), r1(g1), r1(bb1), We1[:D], We1[D:], r1(be1))

    pack = lambda t: lax.bitcast_convert_type(
        t.astype(jnp.bfloat16).reshape(N, _AW, 2), jnp.float32)
    ahp = jnp.concatenate([pack(av), pack(hv)], axis=1)

    perm = jnp.arange(D).reshape(D // 32, 16, 2).transpose(0, 2, 1).reshape(D)
    bp = bv[:, perm]
    params = jnp.concatenate(
        [We2[:, 0][perm], be2, jnp.zeros((_PLEN - D - 1,), jnp.float32)])

    partials = _edge_kernel(row, col, ahp, bp, params)
    partials = partials[:, :, jnp.argsort(perm)]

    out = pl.pallas_call(
        _final_body,
        grid=(N // _BLK,),
        in_specs=[
            pl.BlockSpec((_BLK, D), lambda i: (i, 0)),
            pl.BlockSpec((2, _BLK, D), lambda i: (0, i, 0)),
            pl.BlockSpec((D, D), lambda i: (0, 0)),
            pl.BlockSpec((D, D), lambda i: (0, 0)),
            pl.BlockSpec((1, D), lambda i: (0, 0)),
            pl.BlockSpec((1, D), lambda i: (0, 0)),
            pl.BlockSpec((1, D), lambda i: (0, 0)),
        ],
        out_specs=pl.BlockSpec((_BLK, D), lambda i: (i, 0)),
        out_shape=jax.ShapeDtypeStruct((N, D), jnp.float32),
    )(hv, partials, Wg[:D], Wg[D:], r1(bg), r1(gf), r1(bf))
    return out


# speedup vs baseline: 1.6841x; 1.6841x over previous
import functools

import jax
import jax.numpy as jnp
from jax import lax
from jax.experimental import pallas as pl
from jax.experimental.pallas import tpu as pltpu
from jax.experimental.pallas import tpu_sc as plsc

N = 10000
E = 320000
D = 128
LN_EPS = 1e-5

_BLK = 1000
_C = 40
_NW = 32
_EPT = E // _NW
_NCHUNK = _EPT // _C
_NZC = N // _C
_PLEN = 144


def _prep_body(x_ref, w1_ref, b1_ref, g1_ref, bb1_ref, we1t_ref, we1b_ref,
               be1_ref, ah_ref, b_ref):
    h0 = jnp.dot(x_ref[...], w1_ref[...],
                 preferred_element_type=jnp.float32) + b1_ref[...]
    m = jnp.mean(h0, axis=-1, keepdims=True)
    v = jnp.mean((h0 - m) ** 2, axis=-1, keepdims=True)
    h = jnp.maximum(
        (h0 - m) / jnp.sqrt(v + LN_EPS) * g1_ref[...] + bb1_ref[...], 0.0)
    ah_ref[:, 0:D] = jnp.dot(h, we1t_ref[...],
                             preferred_element_type=jnp.float32)
    ah_ref[:, D:2 * D] = h
    b_ref[...] = jnp.dot(h, we1b_ref[...],
                         preferred_element_type=jnp.float32) + be1_ref[...]


def _final_body(ah_ref, p_ref, wgt_ref, wgb_ref, bg_ref, gf_ref, bf_ref,
                o_ref):
    h = ah_ref[:, D:2 * D]
    aggr = p_ref[0] + p_ref[1]
    g = (jnp.dot(h, wgt_ref[...], preferred_element_type=jnp.float32)
         + jnp.dot(aggr, wgb_ref[...], preferred_element_type=jnp.float32)
         + bg_ref[...])
    gate = 1.0 / (1.0 + jnp.exp(-g))
    hn = gate * aggr + (1.0 - gate) * h
    m = jnp.mean(hn, axis=-1, keepdims=True)
    v = jnp.mean((hn - m) ** 2, axis=-1, keepdims=True)
    o_ref[...] = (hn - m) / jnp.sqrt(v + LN_EPS) * gf_ref[...] + bf_ref[...]


@functools.partial(
    pl.kernel,
    mesh=plsc.VectorSubcoreMesh(core_axis_name="c", subcore_axis_name="s"),
    out_type=jax.ShapeDtypeStruct((2, N, D), jnp.float32),
    scratch_types=[
        pltpu.VMEM((2, _C), jnp.int32),
        pltpu.VMEM((2, _C), jnp.int32),
        pltpu.VMEM((2, _C), jnp.int32),
        pltpu.VMEM((_C, 2 * D), jnp.float32),
        pltpu.VMEM((_C, 2 * D), jnp.float32),
        pltpu.VMEM((_C, D), jnp.float32),
        pltpu.VMEM((_C, D), jnp.float32),
        pltpu.VMEM((_C, D), jnp.float32),
        pltpu.VMEM((_C, D), jnp.float32),
        pltpu.VMEM((_PLEN,), jnp.float32),
        pltpu.VMEM_SHARED((N, D), jnp.float32),
        pltpu.SemaphoreType.DMA,
        pltpu.SemaphoreType.DMA,
        pltpu.SemaphoreType.DMA,
        pltpu.SemaphoreType.DMA,
        pltpu.SemaphoreType.DMA,
        pltpu.SemaphoreType.DMA,
        pltpu.SemaphoreType.DMA,
        pltpu.SemaphoreType.DMA,
        pltpu.SemaphoreType.DMA,
        pltpu.SemaphoreType.DMA,
    ],
)
def _edge_kernel(row_hbm, col_hbm, ah_hbm, b_hbm, par_hbm, out_hbm,
                 rowv, colv, scol, ah0, ah1, bb0, bb1, sc0, sc1, par_buf,
                 aggr, ga0, ga1, gb0, gb1, ss0, ss1, ia0, ia1, ib0, ib1):
    cid = lax.axis_index("c")
    sid = lax.axis_index("s")
    wid = cid * 16 + sid
    ah = (ah0, ah1)
    bb = (bb0, bb1)
    sc = (sc0, sc1)
    ga = (ga0, ga1)
    gb = (gb0, gb1)
    ss = (ss0, ss1)
    ia = (ia0, ia1)
    ib = (ib0, ib1)

    pltpu.sync_copy(par_hbm, par_buf)

    zv = jnp.zeros((16,), jnp.float32)

    def _zrow(r, carry):
        for j in range(D // 16):
            sc0[r, pl.ds(16 * j, 16)] = zv
        return carry

    lax.fori_loop(0, _C, _zrow, 0)
    for k in range((_NZC + 15) // 16):
        ci = sid + 16 * k

        @pl.when(ci < _NZC)
        def _():
            pltpu.sync_copy(sc0, aggr.at[pl.ds(ci * _C, _C)])

    plsc.subcore_barrier()

    e0 = wid * _EPT

    def _issue_gather(p, kk):
        base = e0 + kk * _C
        pltpu.sync_copy(row_hbm.at[pl.ds(base, _C)], rowv.at[p])
        pltpu.sync_copy(col_hbm.at[pl.ds(base, _C)], colv.at[p])
        pltpu.async_copy(ah_hbm.at[rowv.at[p]], ah[p], ga[p])
        pltpu.async_copy(b_hbm.at[colv.at[p]], bb[p], gb[p])

    def _compute(p):
        ahp, bbp, scp = ah[p], bb[p], sc[p]
        lanes = lax.iota(jnp.int32, 16)

        @plsc.parallel_loop(0, _C, unroll=20)
        def _edge(e):
            acc = par_buf[pl.ds(D, 16)]
            for j in range(D // 16):
                a = ahp[e, pl.ds(16 * j, 16)]
                b = bbp[e, pl.ds(16 * j, 16)]
                t = jnp.maximum(a + b, 0.0)
                acc = acc + t * par_buf[pl.ds(16 * j, 16)]
            for sh in (8, 4, 2, 1):
                acc = acc + acc.at[lanes ^ sh].get(mode="promise_in_bounds")
            wv = 1.0 / (1.0 + jnp.exp(-acc))
            for j in range(D // 16):
                scp[e, pl.ds(16 * j, 16)] = (
                    ahp[e, pl.ds(D + 16 * j, 16)] * wv)

    _issue_gather(0, 0)
    _issue_gather(1, 1)

    def _outer(i, carry):
        for p in (0, 1):
            k = 2 * i + p

            @pl.when(k >= 2)
            def _():
                pltpu.make_async_copy(sc[p], aggr.at[scol.at[p]],
                                      ss[p]).wait()

            pltpu.make_async_copy(ah_hbm.at[rowv.at[p]], ah[p], ga[p]).wait()
            pltpu.make_async_copy(b_hbm.at[colv.at[p]], bb[p], gb[p]).wait()
            for off in (0, 16, 24):
                scol[p, pl.ds(off, 16)] = colv[p, pl.ds(off, 16)]

            @pl.when(k + 2 < _NCHUNK)
            def _():
                base = e0 + (k + 2) * _C
                pltpu.async_copy(row_hbm.at[pl.ds(base, _C)], rowv.at[p],
                                 ia[p])
                pltpu.async_copy(col_hbm.at[pl.ds(base, _C)], colv.at[p],
                                 ib[p])

            _compute(p)
            pltpu.async_copy(sc[p], aggr.at[scol.at[p]], ss[p], add=True)

            @pl.when(k + 2 < _NCHUNK)
            def _():
                base = e0 + (k + 2) * _C
                pltpu.make_async_copy(row_hbm.at[pl.ds(base, _C)],
                                      rowv.at[p], ia[p]).wait()
                pltpu.make_async_copy(col_hbm.at[pl.ds(base, _C)],
                                      colv.at[p], ib[p]).wait()
                pltpu.async_copy(ah_hbm.at[rowv.at[p]], ah[p], ga[p])
                pltpu.async_copy(b_hbm.at[colv.at[p]], bb[p], gb[p])
        return carry

    lax.fori_loop(0, _NCHUNK // 2, _outer, 0)
    for p in (0, 1):
        pltpu.make_async_copy(sc[p], aggr.at[scol.at[p]], ss[p]).wait()

    plsc.subcore_barrier()
    for k in range((_NZC + 15) // 16):
        ci = sid + 16 * k

        @pl.when(ci < _NZC)
        def _():
            r0 = ci * _C
            pltpu.sync_copy(aggr.at[pl.ds(r0, _C)],
                            out_hbm.at[cid, pl.ds(r0, _C)])


def kernel(x, edge_index, W1, b1, g1, bb1, We1, be1, We2, be2,
           Wn1, bn1, Wn2, bn2, Wg, bg, gf, bf):
    row = edge_index[0].astype(jnp.int32)
    col = edge_index[1].astype(jnp.int32)
    r1 = lambda a: a.reshape(1, D)

    ah, bt = pl.pallas_call(
        _prep_body,
        grid=(N // _BLK,),
        in_specs=[
            pl.BlockSpec((_BLK, D), lambda i: (i, 0)),
            pl.BlockSpec((D, D), lambda i: (0, 0)),
            pl.BlockSpec((1, D), lambda i: (0, 0)),
            pl.BlockSpec((1, D), lambda i: (0, 0)),
            pl.BlockSpec((1, D), lambda i: (0, 0)),
            pl.BlockSpec((D, D), lambda i: (0, 0)),
            pl.BlockSpec((D, D), lambda i: (0, 0)),
            pl.BlockSpec((1, D), lambda i: (0, 0)),
        ],
        out_specs=[
            pl.BlockSpec((_BLK, 2 * D), lambda i: (i, 0)),
            pl.BlockSpec((_BLK, D), lambda i: (i, 0)),
        ],
        out_shape=[
            jax.ShapeDtypeStruct((N, 2 * D), jnp.float32),
            jax.ShapeDtypeStruct((N, D), jnp.float32),
        ],
    )(x, W1, r1(b1), r1(g1), r1(bb1), We1[:D], We1[D:], r1(be1))

    params = jnp.concatenate(
        [We2[:, 0], be2, jnp.zeros((_PLEN - D - 1,), jnp.float32)])

    partials = _edge_kernel(row, col, ah, bt, params)

    out = pl.pallas_call(
        _final_body,
        grid=(N // _BLK,),
        in_specs=[
            pl.BlockSpec((_BLK, 2 * D), lambda i: (i, 0)),
            pl.BlockSpec((2, _BLK, D), lambda i: (0, i, 0)),
            pl.BlockSpec((D, D), lambda i: (0, 0)),
            pl.BlockSpec((D, D), lambda i: (0, 0)),
            pl.BlockSpec((1, D), lambda i: (0, 0)),
            pl.BlockSpec((1, D), lambda i: (0, 0)),
            pl.BlockSpec((1, D), lambda i: (0, 0)),
        ],
        out_specs=pl.BlockSpec((_BLK, D), lambda i: (i, 0)),
        out_shape=jax.ShapeDtypeStruct((N, D), jnp.float32),
    )(ah, partials, Wg[:D], Wg[D:], r1(bg), r1(gf), r1(bf))
    return out
